# baseline (speedup 1.0000x reference)
import functools
import math

import jax
import jax.numpy as jnp
from jax import lax
from jax.experimental import pallas as pl
from jax.experimental.pallas import tpu as pltpu
from jax.experimental.pallas import tpu_sc as plsc

H = 128
CH = 128
NSUB = 16
NCORE = 2
GRP = 8


def _ceil_to(x, m):
    return (x + m - 1) // m * m


def _mesh():
    return plsc.VectorSubcoreMesh(core_axis_name="c", subcore_axis_name="s",
                                  num_cores=NCORE, num_subcores=NSUB)


@functools.lru_cache(maxsize=None)
def _sc_embed(n_rows_table, b_total):
    cpw = b_total // (NCORE * NSUB * CH)

    @functools.partial(
        pl.kernel,
        out_type=jax.ShapeDtypeStruct((b_total, H), jnp.float32),
        mesh=_mesh(),
        scratch_types=[
            pltpu.VMEM((cpw * CH,), jnp.int32),
            pltpu.VMEM((CH, H), jnp.float32),
            pltpu.VMEM((CH, H), jnp.float32),
        ] + [pltpu.SemaphoreType.DMA] * 4,
    )
    def k(table_hbm, idx_hbm, out_hbm, idx_v, rows0, rows1,
          gm0, gm1, om0, om1):
        c = lax.axis_index("c")
        s = lax.axis_index("s")
        wid = s * NCORE + c
        base = wid * cpw * CH
        rows = (rows0, rows1)
        gsem = (gm0, gm1)
        osem = (om0, om1)
        def gath(j, b):
            return pltpu.make_async_copy(
                table_hbm.at[idx_v.at[pl.ds(j * CH, CH)]], rows[b], gsem[b])

        def copyout(j, b):
            return pltpu.make_async_copy(
                rows[b], out_hbm.at[pl.ds(base + j * CH, CH)], osem[b])

        pltpu.sync_copy(idx_hbm.at[pl.ds(base, cpw * CH)], idx_v)
        gath(0, 0).start()
        for j in range(cpw):
            b = j % 2
            if j + 1 < cpw:
                if j >= 1:
                    copyout(j - 1, 1 - b).wait()
                gath(j + 1, 1 - b).start()
            gath(j, b).wait()
            copyout(j, b).start()
        copyout(cpw - 2, (cpw - 2) % 2).wait()
        copyout(cpw - 1, (cpw - 1) % 2).wait()

    return k


@functools.lru_cache(maxsize=None)
def _sc_counts(nrows, nch_t, nch_q):
    @functools.partial(
        pl.kernel,
        out_type=(
            jax.ShapeDtypeStruct((nrows, H), jnp.float32),
            jax.ShapeDtypeStruct((nrows, H), jnp.float32),
        ),
        mesh=_mesh(),
        scratch_types=[
            pltpu.VMEM_SHARED((nrows, H), jnp.float32),
            pltpu.VMEM((CH,), jnp.int32),
            pltpu.VMEM((CH, H), jnp.float32),
        ],
    )
    def k(dst_t_hbm, dst_q_hbm, ones_hbm, zs_hbm, out_t, out_q,
          acc, dstv, ones_v):
        c = lax.axis_index("c")
        s = lax.axis_index("s")
        rpt = nrows // NSUB
        rlo = s * rpt
        pltpu.sync_copy(ones_hbm, ones_v)

        def run_graph(dst_hbm, out, nch):
            base = s * nch
            pltpu.sync_copy(zs_hbm.at[pl.ds(rlo, rpt)],
                            acc.at[pl.ds(rlo, rpt)])
            plsc.subcore_barrier()

            def chunk(g, carry):
                pltpu.sync_copy(dst_hbm.at[base + g], dstv)
                pltpu.sync_copy(ones_v, acc.at[dstv], add=True)
                return carry

            lax.fori_loop(0, nch, chunk, 0)
            plsc.subcore_barrier()
            pltpu.sync_copy(acc.at[pl.ds(rlo, rpt)],
                            out.at[pl.ds(rlo, rpt)])

        @pl.when(c == 0)
        def _():
            run_graph(dst_t_hbm, out_t, nch_t)

        @pl.when(c == 1)
        def _():
            run_graph(dst_q_hbm, out_q, nch_q)

    return k


@functools.lru_cache(maxsize=None)
def _sc_segsum(nrows, nch_t, nch_q):
    @functools.partial(
        pl.kernel,
        out_type=(
            jax.ShapeDtypeStruct((nrows, H), jnp.float32),
            jax.ShapeDtypeStruct((nrows, H), jnp.float32),
        ),
        mesh=_mesh(),
        scratch_types=[
            pltpu.VMEM_SHARED((nrows, H), jnp.float32),
            pltpu.VMEM((GRP, CH), jnp.int32),
            pltpu.VMEM((GRP, CH), jnp.int32),
            pltpu.VMEM((GRP, CH), jnp.int32),
            pltpu.VMEM((GRP, CH), jnp.int32),
            pltpu.VMEM((CH, H), jnp.float32),
            pltpu.VMEM((CH, H), jnp.float32),
        ] + [pltpu.SemaphoreType.DMA] * 6,
    )
    def k(xt_hbm, src_t_hbm, dst_t_hbm, xq_hbm, src_q_hbm, dst_q_hbm,
          zs_hbm, out_t, out_q, acc, sb0, sb1, db0, db1, r0, r1,
          im0, im1, gm0, gm1, sm0, sm1):
        c = lax.axis_index("c")
        s = lax.axis_index("s")
        sb = (sb0, sb1)
        db = (db0, db1)
        rows = (r0, r1)
        isem = (im0, im1)
        gsem = (gm0, gm1)
        ssem = (sm0, sm1)
        rpt = nrows // NSUB
        rlo = s * rpt

        def prologue():
            pltpu.sync_copy(zs_hbm.at[pl.ds(rlo, rpt)],
                            acc.at[pl.ds(rlo, rpt)])

        def writeout(out):
            plsc.subcore_barrier()
            pltpu.sync_copy(acc.at[pl.ds(rlo, rpt)],
                            out.at[pl.ds(rlo, rpt)])

        def run_ring(x_hbm, src_hbm, dst_hbm, out, nch):
            ng = nch // GRP
            cb = s * nch

            def idxs(o, ob):
                return pltpu.make_async_copy(
                    src_hbm.at[pl.ds(cb + o * GRP, GRP)], sb[ob], isem[ob])

            def idxd(o, ob):
                return pltpu.make_async_copy(
                    dst_hbm.at[pl.ds(cb + o * GRP, GRP)], db[ob], isem[ob])

            def gath(ob, j):
                return pltpu.make_async_copy(
                    x_hbm.at[sb[ob].at[j]], rows[j % 2], gsem[j % 2])

            prologue()
            idxs(0, 0).start()
            idxd(0, 0).start()
            plsc.subcore_barrier()

            def group(o, ob):
                idxs(o, ob).wait()
                idxd(o, ob).wait()
                gath(ob, 0).start()
                for j in range(GRP):
                    if j == 0:
                        @pl.when(o + 1 < ng)
                        def _():
                            idxs(o + 1, 1 - ob).start()
                            idxd(o + 1, 1 - ob).start()

                    gath(ob, j).wait()
                    if j + 1 < GRP:
                        gath(ob, j + 1).start()
                    pltpu.sync_copy(rows[j % 2], acc.at[db[ob].at[j]],
                                    add=True)

            def outer(oo, carry):
                group(2 * oo, 0)
                group(2 * oo + 1, 1)
                return carry

            lax.fori_loop(0, ng // 2, outer, 0)
            writeout(out)

        def run_serial(x_hbm, src_hbm, dst_hbm, out, nch):
            cb = s * nch
            prologue()
            plsc.subcore_barrier()

            def chunk(g, carry):
                pltpu.sync_copy(src_hbm.at[cb + g], sb0.at[0])
                pltpu.sync_copy(dst_hbm.at[cb + g], db0.at[0])
                pltpu.async_copy(x_hbm.at[sb0.at[0]], r0, gm0).wait()
                pltpu.sync_copy(r0, acc.at[db0.at[0]], add=True)
                return carry

            lax.fori_loop(0, nch, chunk, 0)
            writeout(out)

        @pl.when(c == 0)
        def _():
            run_serial(xt_hbm, src_t_hbm, dst_t_hbm, out_t, nch_t)

        @pl.when(c == 1)
        def _():
            run_serial(xq_hbm, src_q_hbm, dst_q_hbm, out_q, nch_q)

    return k


def _sage_update(s, cnt, x, Wl, bl2d, Wr):
    n = x.shape[0]
    R = 400
    assert n % R == 0

    def body(s_ref, c_ref, x_ref, wl_ref, bl_ref, wr_ref, o_ref):
        rinv = 1.0 / jnp.maximum(c_ref[...], 1.0)
        mean = s_ref[...] * rinv
        z = lax.dot_general(mean, wl_ref[...], (((1,), (1,)), ((), ())),
                            preferred_element_type=jnp.float32)
        z = z + bl_ref[...]
        z = z + lax.dot_general(x_ref[...], wr_ref[...],
                                (((1,), (1,)), ((), ())),
                                preferred_element_type=jnp.float32)
        o_ref[...] = jnp.where(z > 0, z, jnp.exp(jnp.minimum(z, 0.0)) - 1.0)

    row = pl.BlockSpec((R, H), lambda i: (i, 0))
    full = pl.BlockSpec((H, H), lambda i: (0, 0))
    return pl.pallas_call(
        body,
        grid=(n // R,),
        in_specs=[row, row, row, full,
                  pl.BlockSpec((1, H), lambda i: (0, 0)), full],
        out_specs=row,
        out_shape=jax.ShapeDtypeStruct((n, H), jnp.float32),
        compiler_params=pltpu.CompilerParams(
            dimension_semantics=("arbitrary",)),
    )(s, cnt, x, Wl, bl2d, Wr)


def _attention(eq, et, mask):
    nq, nt = mask.shape
    R = 200
    assert nq % R == 0
    scale = 1.0 / math.sqrt(H)

    def body(q_ref, t_ref, m_ref, o_ref):
        att = lax.dot_general(q_ref[...], t_ref[...], (((1,), (1,)), ((), ())),
                              preferred_element_type=jnp.float32)
        logits = jnp.where(m_ref[...], att * scale, -1e9)
        mx = jnp.max(logits, axis=1, keepdims=True)
        e = jnp.exp(logits - mx)
        o_ref[...] = e / jnp.sum(e, axis=1, keepdims=True)

    return pl.pallas_call(
        body,
        grid=(nq // R,),
        in_specs=[
            pl.BlockSpec((R, H), lambda i: (i, 0)),
            pl.BlockSpec((nt, H), lambda i: (0, 0)),
            pl.BlockSpec((R, nt), lambda i: (i, 0)),
        ],
        out_specs=pl.BlockSpec((R, nt), lambda i: (i, 0)),
        out_shape=jax.ShapeDtypeStruct((nq, nt), jnp.float32),
        compiler_params=pltpu.CompilerParams(
            dimension_semantics=("parallel",)),
    )(eq, et, mask)


def _pad_edges(ei, n_pad, dummy_row):
    src = ei[0].astype(jnp.int32)
    dst = ei[1].astype(jnp.int32)
    pad = n_pad - src.shape[0]
    if pad:
        src = jnp.concatenate([src, jnp.zeros((pad,), jnp.int32)])
        dst = jnp.concatenate([dst, jnp.full((pad,), dummy_row, jnp.int32)])
    return src.reshape(-1, CH), dst.reshape(-1, CH)


def kernel(target_x, target_edge_index, query_x, query_edge_index, mask, emb,
           Wl0, bl0, Wr0, Wl1, bl1, Wr1, Wl2, bl2, Wr2):
    nt = target_x.shape[0]
    nq = query_x.shape[0]
    et = target_edge_index.shape[1]
    eq = query_edge_index.shape[1]
    dummy = max(nt, nq)
    nrows = _ceil_to(dummy + 1, NSUB * 8)

    b_total = _ceil_to(nt + nq, NCORE * NSUB * CH)
    idx = jnp.concatenate([target_x, query_x]).astype(jnp.int32)
    idx = jnp.concatenate([idx, jnp.zeros((b_total - nt - nq,), jnp.int32)])
    rows = _sc_embed(emb.shape[0], b_total)(emb.astype(jnp.float32), idx)
    xt = rows[:nt]
    xq = rows[nt:nt + nq]

    nch_t = _ceil_to(_ceil_to(et, NSUB * CH) // (NSUB * CH), 2 * GRP)
    nch_q = _ceil_to(_ceil_to(eq, NSUB * CH) // (NSUB * CH), 2 * GRP)
    src_t, dst_t = _pad_edges(target_edge_index, nch_t * NSUB * CH, dummy)
    src_q, dst_q = _pad_edges(query_edge_index, nch_q * NSUB * CH, dummy)

    ones = jnp.ones((CH, H), jnp.float32)
    zs = jnp.zeros((nrows, H), jnp.float32)
    cnt_t, cnt_q = _sc_counts(nrows, nch_t, nch_q)(dst_t, dst_q, ones, zs)
    cnt_t, cnt_q = cnt_t[:nt], cnt_q[:nq]

    seg = _sc_segsum(nrows, nch_t, nch_q)
    for (Wl, bl, Wr) in ((Wl0, bl0, Wr0), (Wl1, bl1, Wr1), (Wl2, bl2, Wr2)):
        s_t, s_q = seg(xt, src_t, dst_t, xq, src_q, dst_q, zs)
        bl2d = bl.reshape(1, H)
        xt = _sage_update(s_t[:nt], cnt_t, xt, Wl, bl2d, Wr)
        xq = _sage_update(s_q[:nq], cnt_q, xq, Wl, bl2d, Wr)

    att = _attention(xq, xt, mask)
    return att[None, ...]


# speedup vs baseline: 1.3430x; 1.3430x over previous
import functools
import math

import jax
import jax.numpy as jnp
from jax import lax
from jax.experimental import pallas as pl
from jax.experimental.pallas import tpu as pltpu
from jax.experimental.pallas import tpu_sc as plsc

H = 128
CH = 128
NSUB = 16
NCORE = 2
CNTW = 128


def _ceil_to(x, m):
    return (x + m - 1) // m * m


def _mesh():
    return plsc.VectorSubcoreMesh(core_axis_name="c", subcore_axis_name="s",
                                  num_cores=NCORE, num_subcores=NSUB)


@functools.lru_cache(maxsize=None)
def _sc_embed(n_rows_table, b_total):
    cpw = b_total // (NCORE * NSUB * CH)

    @functools.partial(
        pl.kernel,
        out_type=jax.ShapeDtypeStruct((b_total, H), jnp.float32),
        mesh=_mesh(),
        scratch_types=[
            pltpu.VMEM((CH,), jnp.int32),
            pltpu.VMEM((CH, H), jnp.float32),
            pltpu.SemaphoreType.DMA,
        ],
    )
    def k(table_hbm, idx_hbm, out_hbm, idx_v, rows_v, sem):
        c = lax.axis_index("c")
        s = lax.axis_index("s")
        wid = s * NCORE + c

        def chunk(g, carry):
            base = (wid * cpw + g) * CH
            pltpu.sync_copy(idx_hbm.at[pl.ds(base, CH)], idx_v)
            pltpu.async_copy(table_hbm.at[idx_v], rows_v, sem).wait()
            pltpu.sync_copy(rows_v, out_hbm.at[pl.ds(base, CH)])
            return carry

        lax.fori_loop(0, cpw, chunk, 0)

    return k


@functools.lru_cache(maxsize=None)
def _sc_counts(nrows, nchunks_t, nchunks_q):
    @functools.partial(
        pl.kernel,
        out_type=(
            jax.ShapeDtypeStruct((nrows, CNTW), jnp.float32),
            jax.ShapeDtypeStruct((nrows, CNTW), jnp.float32),
        ),
        mesh=_mesh(),
        scratch_types=[
            pltpu.VMEM_SHARED((nrows, CNTW), jnp.float32),
            pltpu.VMEM((CH,), jnp.int32),
            pltpu.VMEM((CH, CNTW), jnp.float32),
        ],
    )
    def k(dst_t_hbm, dst_q_hbm, ones_hbm, zc_hbm, out_t, out_q,
          acc, dst_v, ones_v):
        c = lax.axis_index("c")
        s = lax.axis_index("s")
        rpt = nrows // NSUB
        pltpu.sync_copy(ones_hbm, ones_v)

        def do_graph(dst_hbm, out_hbm, nchunks):
            pltpu.sync_copy(zc_hbm.at[pl.ds(s * rpt, rpt)],
                            acc.at[pl.ds(s * rpt, rpt)])
            plsc.subcore_barrier()

            def chunk(g, carry):
                base = (s * nchunks + g) * CH
                pltpu.sync_copy(dst_hbm.at[pl.ds(base, CH)], dst_v)
                pltpu.sync_copy(ones_v, acc.at[dst_v], add=True)
                return carry

            lax.fori_loop(0, nchunks, chunk, 0)
            plsc.subcore_barrier()
            pltpu.sync_copy(acc.at[pl.ds(s * rpt, rpt)],
                            out_hbm.at[pl.ds(s * rpt, rpt)])

        @pl.when(c == 0)
        def _():
            do_graph(dst_t_hbm, out_t, nchunks_t)

        @pl.when(c == 1)
        def _():
            do_graph(dst_q_hbm, out_q, nchunks_q)

    return k


@functools.lru_cache(maxsize=None)
def _sc_segsum(nt, nq, nrows, nchunks_t, nchunks_q):
    @functools.partial(
        pl.kernel,
        out_type=(
            jax.ShapeDtypeStruct((nrows, H), jnp.float32),
            jax.ShapeDtypeStruct((nrows, H), jnp.float32),
        ),
        mesh=_mesh(),
        scratch_types=[
            pltpu.VMEM_SHARED((nrows, H), jnp.float32),
            pltpu.VMEM((CH,), jnp.int32),
            pltpu.VMEM((CH,), jnp.int32),
            pltpu.VMEM((CH, H), jnp.float32),
            pltpu.SemaphoreType.DMA,
        ],
    )
    def k(xt_hbm, src_t_hbm, dst_t_hbm, xq_hbm, src_q_hbm, dst_q_hbm,
          zs_hbm, out_t, out_q, acc, src_v, dst_v, rows_v, sem):
        c = lax.axis_index("c")
        s = lax.axis_index("s")
        rpt = nrows // NSUB

        def do_graph(x_hbm, src_hbm, dst_hbm, out_hbm, nchunks):
            pltpu.sync_copy(zs_hbm.at[pl.ds(s * rpt, rpt)],
                            acc.at[pl.ds(s * rpt, rpt)])
            plsc.subcore_barrier()

            def chunk(g, carry):
                base = (s * nchunks + g) * CH
                pltpu.sync_copy(src_hbm.at[pl.ds(base, CH)], src_v)
                pltpu.sync_copy(dst_hbm.at[pl.ds(base, CH)], dst_v)
                pltpu.async_copy(x_hbm.at[src_v], rows_v, sem).wait()
                pltpu.sync_copy(rows_v, acc.at[dst_v], add=True)
                return carry

            lax.fori_loop(0, nchunks, chunk, 0)
            plsc.subcore_barrier()
            pltpu.sync_copy(acc.at[pl.ds(s * rpt, rpt)],
                            out_hbm.at[pl.ds(s * rpt, rpt)])

        @pl.when(c == 0)
        def _():
            do_graph(xt_hbm, src_t_hbm, dst_t_hbm, out_t, nchunks_t)

        @pl.when(c == 1)
        def _():
            do_graph(xq_hbm, src_q_hbm, dst_q_hbm, out_q, nchunks_q)

    return k


def _sage_update(s, cnt, x, Wl, bl2d, Wr):
    n = x.shape[0]
    R = 400
    assert n % R == 0

    def body(s_ref, c_ref, x_ref, wl_ref, bl_ref, wr_ref, o_ref):
        rinv = 1.0 / jnp.maximum(c_ref[:, 0:1], 1.0)
        mean = s_ref[...] * rinv
        z = lax.dot_general(mean, wl_ref[...], (((1,), (1,)), ((), ())),
                            preferred_element_type=jnp.float32)
        z = z + bl_ref[...]
        z = z + lax.dot_general(x_ref[...], wr_ref[...],
                                (((1,), (1,)), ((), ())),
                                preferred_element_type=jnp.float32)
        o_ref[...] = jnp.where(z > 0, z, jnp.exp(jnp.minimum(z, 0.0)) - 1.0)

    return pl.pallas_call(
        body,
        grid=(n // R,),
        in_specs=[
            pl.BlockSpec((R, H), lambda i: (i, 0)),
            pl.BlockSpec((R, CNTW), lambda i: (i, 0)),
            pl.BlockSpec((R, H), lambda i: (i, 0)),
            pl.BlockSpec((H, H), lambda i: (0, 0)),
            pl.BlockSpec((1, H), lambda i: (0, 0)),
            pl.BlockSpec((H, H), lambda i: (0, 0)),
        ],
        out_specs=pl.BlockSpec((R, H), lambda i: (i, 0)),
        out_shape=jax.ShapeDtypeStruct((n, H), jnp.float32),
        compiler_params=pltpu.CompilerParams(
            dimension_semantics=("arbitrary",)),
    )(s, cnt, x, Wl, bl2d, Wr)


def _attention(eq, et, mask):
    nq, nt = mask.shape
    R = 200
    assert nq % R == 0
    scale = 1.0 / math.sqrt(H)

    def body(q_ref, t_ref, m_ref, o_ref):
        att = lax.dot_general(q_ref[...], t_ref[...], (((1,), (1,)), ((), ())),
                              preferred_element_type=jnp.float32)
        logits = jnp.where(m_ref[...], att * scale, -1e9)
        mx = jnp.max(logits, axis=1, keepdims=True)
        e = jnp.exp(logits - mx)
        o_ref[...] = e / jnp.sum(e, axis=1, keepdims=True)

    return pl.pallas_call(
        body,
        grid=(nq // R,),
        in_specs=[
            pl.BlockSpec((R, H), lambda i: (i, 0)),
            pl.BlockSpec((nt, H), lambda i: (0, 0)),
            pl.BlockSpec((R, nt), lambda i: (i, 0)),
        ],
        out_specs=pl.BlockSpec((R, nt), lambda i: (i, 0)),
        out_shape=jax.ShapeDtypeStruct((nq, nt), jnp.float32),
        compiler_params=pltpu.CompilerParams(
            dimension_semantics=("parallel",)),
    )(eq, et, mask)


def _pad_edges(ei, n_pad, dummy_row):
    src = ei[0].astype(jnp.int32)
    dst = ei[1].astype(jnp.int32)
    pad = n_pad - src.shape[0]
    if pad:
        src = jnp.concatenate([src, jnp.zeros((pad,), jnp.int32)])
        dst = jnp.concatenate([dst, jnp.full((pad,), dummy_row, jnp.int32)])
    return src, dst


def kernel(target_x, target_edge_index, query_x, query_edge_index, mask, emb,
           Wl0, bl0, Wr0, Wl1, bl1, Wr1, Wl2, bl2, Wr2):
    nt = target_x.shape[0]
    nq = query_x.shape[0]
    et = target_edge_index.shape[1]
    eq = query_edge_index.shape[1]
    dummy = max(nt, nq)
    nrows = _ceil_to(dummy + 1, NSUB * 8)

    b_total = _ceil_to(nt + nq, NCORE * NSUB * CH)
    idx = jnp.concatenate([target_x, query_x]).astype(jnp.int32)
    idx = jnp.concatenate([idx, jnp.zeros((b_total - nt - nq,), jnp.int32)])
    rows = _sc_embed(emb.shape[0], b_total)(emb.astype(jnp.float32), idx)
    xt = rows[:nt]
    xq = rows[nt:nt + nq]

    nchunks_t = _ceil_to(et, NSUB * CH) // (NSUB * CH)
    nchunks_q = _ceil_to(eq, NSUB * CH) // (NSUB * CH)
    src_t, dst_t = _pad_edges(target_edge_index, nchunks_t * NSUB * CH, dummy)
    src_q, dst_q = _pad_edges(query_edge_index, nchunks_q * NSUB * CH, dummy)

    ones = jnp.ones((CH, CNTW), jnp.float32)
    zc = jnp.zeros((nrows, CNTW), jnp.float32)
    cnt_t, cnt_q = _sc_counts(nrows, nchunks_t, nchunks_q)(
        dst_t, dst_q, ones, zc)
    cnt_t = cnt_t[:nt]
    cnt_q = cnt_q[:nq]

    zs = jnp.zeros((nrows, H), jnp.float32)
    seg = _sc_segsum(nt, nq, nrows, nchunks_t, nchunks_q)
    for (Wl, bl, Wr) in ((Wl0, bl0, Wr0), (Wl1, bl1, Wr1), (Wl2, bl2, Wr2)):
        s_t, s_q = seg(xt, src_t, dst_t, xq, src_q, dst_q, zs)
        bl2d = bl.reshape(1, H)
        xt = _sage_update(s_t[:nt], cnt_t, xt, Wl, bl2d, Wr)
        xq = _sage_update(s_q[:nq], cnt_q, xq, Wl, bl2d, Wr)

    att = _attention(xq, xt, mask)
    return att[None, ...]
